# baseline (speedup 1.0000x reference)
import jax
import jax.numpy as jnp
from jax import lax
from jax.experimental import pallas as pl
from jax.experimental.pallas import tpu as pltpu
from jax.experimental.pallas import tpu_sc as plsc

B, N, L = 1, 128, 384
D_INIT, D_MSA, D_PAIR, D_STATE = 48, 256, 128, 32
NBIN = 65
NSEQ = 22

_NW = 32
_ROWS_PER_W = L // _NW


_PACKED_ROW = D_PAIR // 2
_RIGHT_OFF = NSEQ * _PACKED_ROW
_POS_OFF = 2 * NSEQ * _PACKED_ROW
_TBL_ROWS = 2 * NSEQ + NBIN
_TBL_WORDS = _TBL_ROWS * _PACKED_ROW


def _sc_body(seq_hbm, idx_hbm, left_hbm, right_hbm, pos_hbm, sttbl_hbm,
             pair_out, statet_out,
             seq_v, idx_v, raw_v, tbl_v, sttbl_v, combo_v,
             rowbuf0, rowbuf1, stbuf_t, sem0, sem1):
    cid = lax.axis_index("c")
    sid = lax.axis_index("s")
    w = sid * 2 + cid
    base = w * _ROWS_PER_W

    pltpu.sync_copy(seq_hbm, seq_v)
    pltpu.sync_copy(idx_hbm, idx_v)
    pltpu.sync_copy(left_hbm, raw_v.at[pl.ds(0, NSEQ)])
    pltpu.sync_copy(right_hbm, raw_v.at[pl.ds(NSEQ, NSEQ)])
    pltpu.sync_copy(pos_hbm, raw_v.at[pl.ds(2 * NSEQ, NBIN)])
    pltpu.sync_copy(sttbl_hbm, sttbl_v)

    iota = lax.iota(jnp.int32, 16)

    @plsc.parallel_loop(0, _TBL_ROWS, unroll=2)
    def pbody(r):
        for cc in range(D_PAIR // 32):
            a = raw_v[r, pl.ds(32 * cc, 16)]
            b = raw_v[r, pl.ds(32 * cc + 16, 16)]
            pk = plsc.pack(a, b, format=plsc.PackFormat.INTERLEAVED)
            tbl_v[pl.ds(r * _PACKED_ROW + 16 * cc, 16)] = plsc.bitcast(
                pk, jnp.float32)

    @pl.when(w < 3)
    def _():
        for jb in range(8):
            sjv = seq_v[0, pl.ds(w * 128 + jb * 16, 16)]
            for f in range(D_STATE):
                stbuf_t[f, pl.ds(jb * 16, 16)] = plsc.load_gather(
                    sttbl_v, [sjv, jnp.full((16,), f, jnp.int32)])
        pltpu.sync_copy(stbuf_t, statet_out.at[:, pl.ds(w * 128, 128)])

    bufs = (rowbuf0, rowbuf1)
    sems = (sem0, sem1)
    pending = [None, None]
    for rr in range(_ROWS_PER_W):
        i = base + rr
        k = rr % 2
        if pending[k] is not None:
            pending[k].wait()
        buf = bufs[k]
        i16 = jnp.full((16,), i, jnp.int32)
        zero16 = jnp.zeros((16,), jnp.int32)
        si = plsc.load_gather(seq_v, [zero16, i16]) * _PACKED_ROW
        di = plsc.load_gather(idx_v, [zero16, i16])
        rrow = []
        for cc in range(D_PAIR // 32):
            rw = plsc.load_gather(tbl_v, [_RIGHT_OFF + si + iota + 16 * cc])
            ra, rb = plsc.unpack(plsc.bitcast(rw, jnp.bfloat16),
                                 format=plsc.PackFormat.INTERLEAVED)
            rrow += [ra, rb]

        @plsc.parallel_loop(0, L // 16, unroll=2)
        def cbody(jb, di=di):
            sjv = seq_v[0, pl.ds(jb * 16, 16)] * _PACKED_ROW
            djv = idx_v[0, pl.ds(jb * 16, 16)]
            pidx = (jnp.clip(djv - di + 32, 0, NBIN - 1) * _PACKED_ROW
                    + _POS_OFF)
            combo_v[pl.ds(jb * 16, 16)] = (pidx << 16) | sjv

        @plsc.parallel_loop(0, L, unroll=2)
        def jbody(j, buf=buf, rrow=rrow):
            j16 = jnp.full((16,), j, jnp.int32)
            cw = plsc.load_gather(combo_v, [j16])
            sj = cw & 0xFFFF
            pidx = lax.shift_right_logical(cw, 16)
            for cc in range(D_PAIR // 32):
                lw = plsc.load_gather(tbl_v, [sj + iota + 16 * cc])
                pw = plsc.load_gather(tbl_v, [pidx + iota + 16 * cc])
                lp = (plsc.bitcast(lw, jnp.bfloat16)
                      + plsc.bitcast(pw, jnp.bfloat16))
                a, b = plsc.unpack(lp, format=plsc.PackFormat.INTERLEAVED)
                buf[j, pl.ds(32 * cc, 16)] = a + rrow[2 * cc]
                buf[j, pl.ds(32 * cc + 16, 16)] = b + rrow[2 * cc + 1]

        pending[k] = pltpu.async_copy(buf, pair_out.at[i], sems[k])
    pending[0].wait()
    pending[1].wait()


def _sc_pair_state(seq, idx, emb_left, emb_right, pos_emb, emb_state):
    mesh = plsc.VectorSubcoreMesh(core_axis_name="c", subcore_axis_name="s")
    kern = pl.kernel(
        _sc_body,
        out_type=[
            jax.ShapeDtypeStruct((L, L, D_PAIR), jnp.float32),
            jax.ShapeDtypeStruct((D_STATE, L), jnp.float32),
        ],
        mesh=mesh,
        compiler_params=pltpu.CompilerParams(needs_layout_passes=False),
        scratch_types=[
            pltpu.VMEM((1, L), jnp.int32),
            pltpu.VMEM((1, L), jnp.int32),
            pltpu.VMEM((_TBL_ROWS, D_PAIR), jnp.float32),
            pltpu.VMEM((_TBL_WORDS,), jnp.float32),
            pltpu.VMEM((NSEQ, D_STATE), jnp.float32),
            pltpu.VMEM((L,), jnp.int32),
            pltpu.VMEM((L, D_PAIR), jnp.float32),
            pltpu.VMEM((L, D_PAIR), jnp.float32),
            pltpu.VMEM((D_STATE, 128), jnp.float32),
            pltpu.SemaphoreType.DMA,
            pltpu.SemaphoreType.DMA,
        ],
    )
    return kern(seq, idx, emb_left, emb_right, pos_emb, emb_state)


_N_BLK = 8


def _tc_body(seq_ref, msa_ref, w_ref, b_ref, q_ref, out_ref, qrow):
    n = pl.program_id(0)

    @pl.when(n == 0)
    def _():
        seq = seq_ref[...]
        onehot_t = (jnp.broadcast_to(seq, (NSEQ, L))
                    == lax.broadcasted_iota(jnp.int32, (NSEQ, L), 0)
                    ).astype(jnp.float32)
        qrow[...] = (lax.dot_general(onehot_t, q_ref[...],
                                     (((0,), (0,)), ((), ())),
                                     preferred_element_type=jnp.float32)
                     + b_ref[...])

    for b in range(_N_BLK):
        x = msa_ref[b]
        y = lax.dot_general(x, w_ref[...], (((0,), (0,)), ((), ())),
                            preferred_element_type=jnp.float32)
        out_ref[b] = y + qrow[...]


def _tc_msa(seq2d, msa3t, emb_Wt, emb_b, emb_q):
    grid = (N // _N_BLK,)
    return pl.pallas_call(
        _tc_body,
        grid=grid,
        in_specs=[
            pl.BlockSpec((1, L), lambda n: (0, 0)),
            pl.BlockSpec((_N_BLK, D_INIT, L), lambda n: (n, 0, 0)),
            pl.BlockSpec((D_INIT, D_MSA), lambda n: (0, 0)),
            pl.BlockSpec((1, D_MSA), lambda n: (0, 0)),
            pl.BlockSpec((NSEQ, D_MSA), lambda n: (0, 0)),
        ],
        out_specs=pl.BlockSpec((_N_BLK, L, D_MSA), lambda n: (n, 0, 0)),
        out_shape=jax.ShapeDtypeStruct((N, L, D_MSA), jnp.float32),
        scratch_shapes=[pltpu.VMEM((L, D_MSA), jnp.float32)],
    )(seq2d, msa3t, emb_Wt, emb_b, emb_q)


@jax.jit
def kernel(msa, seq, idx, emb_W, emb_b, emb_q, emb_left, emb_right,
           emb_state, pos_emb):
    seq2 = seq.reshape(1, L).astype(jnp.int32)
    idx2 = idx.reshape(1, L).astype(jnp.int32)

    msa_e = _tc_msa(seq2,
                    msa.reshape(N, L, D_INIT).transpose(0, 2, 1),
                    emb_W.T, emb_b.reshape(1, D_MSA), emb_q)
    pair, statet = _sc_pair_state(seq2, idx2, emb_left, emb_right, pos_emb,
                                  emb_state)

    return (msa_e.reshape(B, N, L, D_MSA),
            pair.reshape(B, L, L, D_PAIR),
            statet.T.reshape(B, L, D_STATE))


# speedup vs baseline: 1.0354x; 1.0354x over previous
import jax
import jax.numpy as jnp
from jax import lax
from jax.experimental import pallas as pl
from jax.experimental.pallas import tpu as pltpu
from jax.experimental.pallas import tpu_sc as plsc

B, N, L = 1, 128, 384
D_INIT, D_MSA, D_PAIR, D_STATE = 48, 256, 128, 32
NBIN = 65
NSEQ = 22

_NW = 32
_ROWS_PER_W = L // _NW


_PACKED_ROW = D_PAIR // 2
_RIGHT_OFF = NSEQ * _PACKED_ROW
_POS_OFF = 2 * NSEQ * _PACKED_ROW
_TBL_ROWS = 2 * NSEQ + NBIN
_TBL_WORDS = _TBL_ROWS * _PACKED_ROW


def _sc_body(seq_hbm, idx_hbm, rawtbl_hbm, sttbl_hbm,
             pair_out, statet_out,
             seq_v, idx_v, raw_v, tbl_v, sttbl_v, combo_v,
             rowbuf0, rowbuf1, stbuf_t, sem0, sem1):
    cid = lax.axis_index("c")
    sid = lax.axis_index("s")
    w = sid * 2 + cid
    base = w * _ROWS_PER_W

    pltpu.sync_copy(seq_hbm, seq_v)
    pltpu.sync_copy(idx_hbm, idx_v)
    pltpu.sync_copy(rawtbl_hbm, raw_v)
    pltpu.sync_copy(sttbl_hbm, sttbl_v)

    iota = lax.iota(jnp.int32, 16)

    @plsc.parallel_loop(0, _TBL_ROWS, unroll=2)
    def pbody(r):
        for cc in range(D_PAIR // 32):
            a = raw_v[r, pl.ds(32 * cc, 16)]
            b = raw_v[r, pl.ds(32 * cc + 16, 16)]
            pk = plsc.pack(a, b, format=plsc.PackFormat.INTERLEAVED)
            tbl_v[pl.ds(r * _PACKED_ROW + 16 * cc, 16)] = plsc.bitcast(
                pk, jnp.float32)

    @pl.when(w < 3)
    def _():
        for jb in range(8):
            sjv = seq_v[pl.ds(w * 128 + jb * 16, 16)] * D_STATE
            for f in range(D_STATE):
                stbuf_t[f, pl.ds(jb * 16, 16)] = plsc.load_gather(
                    sttbl_v, [sjv + f])
        pltpu.sync_copy(stbuf_t, statet_out.at[:, pl.ds(w * 128, 128)])

    bufs = (rowbuf0, rowbuf1)
    sems = (sem0, sem1)
    pending = [None, None]
    for rr in range(_ROWS_PER_W):
        i = base + rr
        k = rr % 2
        if pending[k] is not None:
            pending[k].wait()
        buf = bufs[k]
        i16 = jnp.full((16,), i, jnp.int32)
        si = plsc.load_gather(seq_v, [i16]) * _PACKED_ROW
        di = plsc.load_gather(idx_v, [i16])
        rrow = []
        for cc in range(D_PAIR // 32):
            rw = plsc.load_gather(tbl_v, [_RIGHT_OFF + si + iota + 16 * cc])
            ra, rb = plsc.unpack(plsc.bitcast(rw, jnp.bfloat16),
                                 format=plsc.PackFormat.INTERLEAVED)
            rrow += [ra, rb]

        @plsc.parallel_loop(0, L // 16, unroll=2)
        def cbody(jb, di=di):
            sjv = seq_v[pl.ds(jb * 16, 16)] * _PACKED_ROW
            djv = idx_v[pl.ds(jb * 16, 16)]
            pidx = (jnp.clip(djv - di + 32, 0, NBIN - 1) * _PACKED_ROW
                    + _POS_OFF)
            combo_v[pl.ds(jb * 16, 16)] = (pidx << 16) | sjv

        @plsc.parallel_loop(0, L, unroll=2)
        def jbody(j, buf=buf, rrow=rrow):
            j16 = jnp.full((16,), j, jnp.int32)
            cw = plsc.load_gather(combo_v, [j16])
            sj = cw & 0xFFFF
            pidx = lax.shift_right_logical(cw, 16)
            for cc in range(D_PAIR // 32):
                lw = plsc.load_gather(tbl_v, [sj + iota + 16 * cc])
                pw = plsc.load_gather(tbl_v, [pidx + iota + 16 * cc])
                lp = (plsc.bitcast(lw, jnp.bfloat16)
                      + plsc.bitcast(pw, jnp.bfloat16))
                a, b = plsc.unpack(lp, format=plsc.PackFormat.INTERLEAVED)
                buf[j, pl.ds(32 * cc, 16)] = a + rrow[2 * cc]
                buf[j, pl.ds(32 * cc + 16, 16)] = b + rrow[2 * cc + 1]

        pending[k] = pltpu.async_copy(buf, pair_out.at[i], sems[k])
    pending[0].wait()
    pending[1].wait()


def _sc_pair_state(seq, idx, emb_left, emb_right, pos_emb, emb_state):
    mesh = plsc.VectorSubcoreMesh(core_axis_name="c", subcore_axis_name="s")
    kern = pl.kernel(
        _sc_body,
        out_type=[
            jax.ShapeDtypeStruct((L, L, D_PAIR), jnp.float32),
            jax.ShapeDtypeStruct((D_STATE, L), jnp.float32),
        ],
        mesh=mesh,
        compiler_params=pltpu.CompilerParams(needs_layout_passes=False),
        scratch_types=[
            pltpu.VMEM((L,), jnp.int32),
            pltpu.VMEM((L,), jnp.int32),
            pltpu.VMEM((_TBL_ROWS, D_PAIR), jnp.float32),
            pltpu.VMEM((_TBL_WORDS,), jnp.float32),
            pltpu.VMEM((NSEQ * D_STATE,), jnp.float32),
            pltpu.VMEM((L,), jnp.int32),
            pltpu.VMEM((L, D_PAIR), jnp.float32),
            pltpu.VMEM((L, D_PAIR), jnp.float32),
            pltpu.VMEM((D_STATE, 128), jnp.float32),
            pltpu.SemaphoreType.DMA,
            pltpu.SemaphoreType.DMA,
        ],
    )
    rawtbl = jnp.concatenate([emb_left, emb_right, pos_emb])
    return kern(seq.reshape(L), idx.reshape(L), rawtbl,
                emb_state.reshape(-1))


_N_BLK = 8


def _tc_body(seq_ref, msa_ref, w_ref, b_ref, q_ref, out_ref, qrow):
    n = pl.program_id(0)

    @pl.when(n == 0)
    def _():
        seq = seq_ref[...]
        onehot_t = (jnp.broadcast_to(seq, (NSEQ, L))
                    == lax.broadcasted_iota(jnp.int32, (NSEQ, L), 0)
                    ).astype(jnp.float32)
        qrow[...] = (lax.dot_general(onehot_t, q_ref[...],
                                     (((0,), (0,)), ((), ())),
                                     preferred_element_type=jnp.float32)
                     + b_ref[...])

    for b in range(_N_BLK):
        x = msa_ref[b]
        y = lax.dot_general(x, w_ref[...], (((0,), (0,)), ((), ())),
                            preferred_element_type=jnp.float32)
        out_ref[b] = y + qrow[...]


def _tc_msa(seq2d, msa3t, emb_Wt, emb_b, emb_q):
    grid = (N // _N_BLK,)
    return pl.pallas_call(
        _tc_body,
        grid=grid,
        in_specs=[
            pl.BlockSpec((1, L), lambda n: (0, 0)),
            pl.BlockSpec((_N_BLK, D_INIT, L), lambda n: (n, 0, 0)),
            pl.BlockSpec((D_INIT, D_MSA), lambda n: (0, 0)),
            pl.BlockSpec((1, D_MSA), lambda n: (0, 0)),
            pl.BlockSpec((NSEQ, D_MSA), lambda n: (0, 0)),
        ],
        out_specs=pl.BlockSpec((_N_BLK, L, D_MSA), lambda n: (n, 0, 0)),
        out_shape=jax.ShapeDtypeStruct((N, L, D_MSA), jnp.float32),
        scratch_shapes=[pltpu.VMEM((L, D_MSA), jnp.float32)],
    )(seq2d, msa3t, emb_Wt, emb_b, emb_q)


@jax.jit
def kernel(msa, seq, idx, emb_W, emb_b, emb_q, emb_left, emb_right,
           emb_state, pos_emb):
    seq2 = seq.reshape(1, L).astype(jnp.int32)
    idx2 = idx.reshape(1, L).astype(jnp.int32)

    msa_e = _tc_msa(seq2,
                    msa.reshape(N, L, D_INIT).transpose(0, 2, 1),
                    emb_W.T, emb_b.reshape(1, D_MSA), emb_q)
    pair, statet = _sc_pair_state(seq2, idx2, emb_left, emb_right, pos_emb,
                                  emb_state)

    return (msa_e.reshape(B, N, L, D_MSA),
            pair.reshape(B, L, L, D_PAIR),
            statet.T.reshape(B, L, D_STATE))


# speedup vs baseline: 1.0592x; 1.0229x over previous
import jax
import jax.numpy as jnp
from jax import lax
from jax.experimental import pallas as pl
from jax.experimental.pallas import tpu as pltpu
from jax.experimental.pallas import tpu_sc as plsc

B, N, L = 1, 128, 384
D_INIT, D_MSA, D_PAIR, D_STATE = 48, 256, 128, 32
NBIN = 65
NSEQ = 22

_NW = 32
_ROWS_PER_W = L // _NW


_PACKED_ROW = D_PAIR // 2
_RIGHT_OFF = NSEQ * _PACKED_ROW
_POS_OFF = 2 * NSEQ * _PACKED_ROW
_TBL_ROWS = 2 * NSEQ + NBIN
_TBL_WORDS = _TBL_ROWS * _PACKED_ROW


def _sc_body(seq_hbm, idx_hbm, rawtbl_hbm, sttbl_hbm,
             pair_out, statet_out,
             seq_v, idx_v, raw_v, tbl_v, sttbl_v, combo_v,
             rowbuf0, rowbuf1, stbuf_t, sem0, sem1):
    cid = lax.axis_index("c")
    sid = lax.axis_index("s")
    w = sid * 2 + cid
    base = w * _ROWS_PER_W

    pltpu.sync_copy(seq_hbm, seq_v)
    pltpu.sync_copy(idx_hbm, idx_v)
    pltpu.sync_copy(rawtbl_hbm, raw_v)
    pltpu.sync_copy(sttbl_hbm, sttbl_v)

    iota = lax.iota(jnp.int32, 16)

    @plsc.parallel_loop(0, _TBL_ROWS, unroll=2)
    def pbody(r):
        for cc in range(D_PAIR // 32):
            a = raw_v[r, pl.ds(32 * cc, 16)]
            b = raw_v[r, pl.ds(32 * cc + 16, 16)]
            pk = plsc.pack(a, b, format=plsc.PackFormat.INTERLEAVED)
            tbl_v[pl.ds(r * _PACKED_ROW + 16 * cc, 16)] = plsc.bitcast(
                pk, jnp.float32)

    @pl.when(w < 3)
    def _():
        for jb in range(8):
            sjv = seq_v[pl.ds(w * 128 + jb * 16, 16)] * D_STATE
            for f in range(D_STATE):
                stbuf_t[f, pl.ds(jb * 16, 16)] = plsc.load_gather(
                    sttbl_v, [sjv + f])
        pltpu.sync_copy(stbuf_t, statet_out.at[:, pl.ds(w * 128, 128)])

    bufs = (rowbuf0, rowbuf1)
    sems = (sem0, sem1)
    pending = [None, None]
    for rr in range(_ROWS_PER_W):
        i = base + rr
        k = rr % 2
        if pending[k] is not None:
            pending[k].wait()
        buf = bufs[k]
        i16 = jnp.full((16,), i, jnp.int32)
        si = plsc.load_gather(seq_v, [i16]) * _PACKED_ROW
        di = plsc.load_gather(idx_v, [i16])
        rrow = []
        for cc in range(D_PAIR // 32):
            rw = plsc.load_gather(tbl_v, [_RIGHT_OFF + si + iota + 16 * cc])
            ra, rb = plsc.unpack(plsc.bitcast(rw, jnp.bfloat16),
                                 format=plsc.PackFormat.INTERLEAVED)
            rrow += [ra, rb]

        @plsc.parallel_loop(0, L // 16, unroll=2)
        def cbody(jb, di=di):
            sjv = seq_v[pl.ds(jb * 16, 16)] * _PACKED_ROW
            djv = idx_v[pl.ds(jb * 16, 16)]
            pidx = (jnp.clip(djv - di + 32, 0, NBIN - 1) * _PACKED_ROW
                    + _POS_OFF)
            combo_v[pl.ds(jb * 16, 16)] = (pidx << 16) | sjv

        @plsc.parallel_loop(0, L, unroll=3)
        def jbody(j, buf=buf, rrow=rrow):
            j16 = jnp.full((16,), j, jnp.int32)
            cw = plsc.load_gather(combo_v, [j16])
            sj = cw & 0xFFFF
            pidx = lax.shift_right_logical(cw, 16)
            for cc in range(D_PAIR // 32):
                lw = plsc.load_gather(tbl_v, [sj + iota + 16 * cc])
                pw = plsc.load_gather(tbl_v, [pidx + iota + 16 * cc])
                lp = (plsc.bitcast(lw, jnp.bfloat16)
                      + plsc.bitcast(pw, jnp.bfloat16))
                a, b = plsc.unpack(lp, format=plsc.PackFormat.INTERLEAVED)
                buf[j, pl.ds(32 * cc, 16)] = a + rrow[2 * cc]
                buf[j, pl.ds(32 * cc + 16, 16)] = b + rrow[2 * cc + 1]

        pending[k] = pltpu.async_copy(buf, pair_out.at[i], sems[k])
    pending[0].wait()
    pending[1].wait()


def _sc_pair_state(seq, idx, emb_left, emb_right, pos_emb, emb_state):
    mesh = plsc.VectorSubcoreMesh(core_axis_name="c", subcore_axis_name="s")
    kern = pl.kernel(
        _sc_body,
        out_type=[
            jax.ShapeDtypeStruct((L, L, D_PAIR), jnp.float32),
            jax.ShapeDtypeStruct((D_STATE, L), jnp.float32),
        ],
        mesh=mesh,
        compiler_params=pltpu.CompilerParams(needs_layout_passes=False),
        scratch_types=[
            pltpu.VMEM((L,), jnp.int32),
            pltpu.VMEM((L,), jnp.int32),
            pltpu.VMEM((_TBL_ROWS, D_PAIR), jnp.float32),
            pltpu.VMEM((_TBL_WORDS,), jnp.float32),
            pltpu.VMEM((NSEQ * D_STATE,), jnp.float32),
            pltpu.VMEM((L,), jnp.int32),
            pltpu.VMEM((L, D_PAIR), jnp.float32),
            pltpu.VMEM((L, D_PAIR), jnp.float32),
            pltpu.VMEM((D_STATE, 128), jnp.float32),
            pltpu.SemaphoreType.DMA,
            pltpu.SemaphoreType.DMA,
        ],
    )
    rawtbl = jnp.concatenate([emb_left, emb_right, pos_emb])
    return kern(seq.reshape(L), idx.reshape(L), rawtbl,
                emb_state.reshape(-1))


_N_BLK = 8


def _tc_body(seq_ref, msa_ref, w_ref, b_ref, q_ref, out_ref, qrow):
    n = pl.program_id(0)

    @pl.when(n == 0)
    def _():
        seq = seq_ref[...]
        onehot_t = (jnp.broadcast_to(seq, (NSEQ, L))
                    == lax.broadcasted_iota(jnp.int32, (NSEQ, L), 0)
                    ).astype(jnp.float32)
        qrow[...] = (lax.dot_general(onehot_t, q_ref[...],
                                     (((0,), (0,)), ((), ())),
                                     preferred_element_type=jnp.float32)
                     + b_ref[...])

    for b in range(_N_BLK):
        x = msa_ref[b]
        y = lax.dot_general(x, w_ref[...], (((0,), (0,)), ((), ())),
                            preferred_element_type=jnp.float32)
        out_ref[b] = y + qrow[...]


def _tc_msa(seq2d, msa3t, emb_Wt, emb_b, emb_q):
    grid = (N // _N_BLK,)
    return pl.pallas_call(
        _tc_body,
        grid=grid,
        in_specs=[
            pl.BlockSpec((1, L), lambda n: (0, 0)),
            pl.BlockSpec((_N_BLK, D_INIT, L), lambda n: (n, 0, 0)),
            pl.BlockSpec((D_INIT, D_MSA), lambda n: (0, 0)),
            pl.BlockSpec((1, D_MSA), lambda n: (0, 0)),
            pl.BlockSpec((NSEQ, D_MSA), lambda n: (0, 0)),
        ],
        out_specs=pl.BlockSpec((_N_BLK, L, D_MSA), lambda n: (n, 0, 0)),
        out_shape=jax.ShapeDtypeStruct((N, L, D_MSA), jnp.float32),
        scratch_shapes=[pltpu.VMEM((L, D_MSA), jnp.float32)],
    )(seq2d, msa3t, emb_Wt, emb_b, emb_q)


@jax.jit
def kernel(msa, seq, idx, emb_W, emb_b, emb_q, emb_left, emb_right,
           emb_state, pos_emb):
    seq2 = seq.reshape(1, L).astype(jnp.int32)
    idx2 = idx.reshape(1, L).astype(jnp.int32)

    msa_e = _tc_msa(seq2,
                    msa.reshape(N, L, D_INIT).transpose(0, 2, 1),
                    emb_W.T, emb_b.reshape(1, D_MSA), emb_q)
    pair, statet = _sc_pair_state(seq2, idx2, emb_left, emb_right, pos_emb,
                                  emb_state)

    return (msa_e.reshape(B, N, L, D_MSA),
            pair.reshape(B, L, L, D_PAIR),
            statet.T.reshape(B, L, D_STATE))


# speedup vs baseline: 1.0623x; 1.0029x over previous
import jax
import jax.numpy as jnp
from jax import lax
from jax.experimental import pallas as pl
from jax.experimental.pallas import tpu as pltpu
from jax.experimental.pallas import tpu_sc as plsc

B, N, L = 1, 128, 384
D_INIT, D_MSA, D_PAIR, D_STATE = 48, 256, 128, 32
NBIN = 65
NSEQ = 22

_NW = 32
_ROWS_PER_W = L // _NW


_PACKED_ROW = D_PAIR // 2
_RIGHT_OFF = NSEQ * _PACKED_ROW
_POS_OFF = 2 * NSEQ * _PACKED_ROW
_TBL_ROWS = 2 * NSEQ + NBIN
_TBL_WORDS = _TBL_ROWS * _PACKED_ROW


def _sc_body(seq_hbm, idx_hbm, rawtbl_hbm, sttbl_hbm,
             pair_out, statet_out,
             seq_v, idx_v, raw_v, tbl_v, sttbl_v, combo_v,
             rowbuf0, rowbuf1, stbuf_t, sem0, sem1):
    cid = lax.axis_index("c")
    sid = lax.axis_index("s")
    w = sid * 2 + cid
    base = w * _ROWS_PER_W

    pltpu.sync_copy(seq_hbm, seq_v)
    pltpu.sync_copy(idx_hbm, idx_v)
    pltpu.sync_copy(rawtbl_hbm, raw_v)
    pltpu.sync_copy(sttbl_hbm, sttbl_v)

    iota = lax.iota(jnp.int32, 16)

    @plsc.parallel_loop(0, _TBL_ROWS, unroll=2)
    def pbody(r):
        for cc in range(D_PAIR // 32):
            a = raw_v[r, pl.ds(32 * cc, 16)]
            b = raw_v[r, pl.ds(32 * cc + 16, 16)]
            pk = plsc.pack(a, b, format=plsc.PackFormat.INTERLEAVED)
            tbl_v[pl.ds(r * _PACKED_ROW + 16 * cc, 16)] = plsc.bitcast(
                pk, jnp.float32)

    @pl.when(w < 3)
    def _():
        for jb in range(8):
            sjv = seq_v[pl.ds(w * 128 + jb * 16, 16)] * D_STATE
            for f in range(D_STATE):
                stbuf_t[f, pl.ds(jb * 16, 16)] = plsc.load_gather(
                    sttbl_v, [sjv + f])
        pltpu.sync_copy(stbuf_t, statet_out.at[:, pl.ds(w * 128, 128)])

    bufs = (rowbuf0, rowbuf1)
    sems = (sem0, sem1)
    pending = [None, None]
    for rr in range(_ROWS_PER_W):
        i = base + rr
        k = rr % 2
        if pending[k] is not None:
            pending[k].wait()
        buf = bufs[k]
        i16 = jnp.full((16,), i, jnp.int32)
        si = plsc.load_gather(seq_v, [i16]) * _PACKED_ROW
        di = plsc.load_gather(idx_v, [i16])
        rrow = []
        for cc in range(D_PAIR // 32):
            rw = plsc.load_gather(tbl_v, [_RIGHT_OFF + si + iota + 16 * cc])
            ra, rb = plsc.unpack(plsc.bitcast(rw, jnp.bfloat16),
                                 format=plsc.PackFormat.INTERLEAVED)
            rrow += [ra, rb]

        @plsc.parallel_loop(0, L // 16, unroll=2)
        def cbody(jb, di=di):
            sjv = seq_v[pl.ds(jb * 16, 16)] * _PACKED_ROW
            djv = idx_v[pl.ds(jb * 16, 16)]
            pidx = (jnp.clip(djv - di + 32, 0, NBIN - 1) * _PACKED_ROW
                    + _POS_OFF)
            combo_v[pl.ds(jb * 16, 16)] = (pidx << 16) | sjv

        @plsc.parallel_loop(0, L, unroll=4)
        def jbody(j, buf=buf, rrow=rrow):
            j16 = jnp.full((16,), j, jnp.int32)
            cw = plsc.load_gather(combo_v, [j16])
            sj = cw & 0xFFFF
            pidx = lax.shift_right_logical(cw, 16)
            for cc in range(D_PAIR // 32):
                lw = plsc.load_gather(tbl_v, [sj + iota + 16 * cc])
                pw = plsc.load_gather(tbl_v, [pidx + iota + 16 * cc])
                lp = (plsc.bitcast(lw, jnp.bfloat16)
                      + plsc.bitcast(pw, jnp.bfloat16))
                a, b = plsc.unpack(lp, format=plsc.PackFormat.INTERLEAVED)
                buf[j, pl.ds(32 * cc, 16)] = a + rrow[2 * cc]
                buf[j, pl.ds(32 * cc + 16, 16)] = b + rrow[2 * cc + 1]

        pending[k] = pltpu.async_copy(buf, pair_out.at[i], sems[k])
    pending[0].wait()
    pending[1].wait()


def _sc_pair_state(seq, idx, emb_left, emb_right, pos_emb, emb_state):
    mesh = plsc.VectorSubcoreMesh(core_axis_name="c", subcore_axis_name="s")
    kern = pl.kernel(
        _sc_body,
        out_type=[
            jax.ShapeDtypeStruct((L, L, D_PAIR), jnp.float32),
            jax.ShapeDtypeStruct((D_STATE, L), jnp.float32),
        ],
        mesh=mesh,
        compiler_params=pltpu.CompilerParams(needs_layout_passes=False),
        scratch_types=[
            pltpu.VMEM((L,), jnp.int32),
            pltpu.VMEM((L,), jnp.int32),
            pltpu.VMEM((_TBL_ROWS, D_PAIR), jnp.float32),
            pltpu.VMEM((_TBL_WORDS,), jnp.float32),
            pltpu.VMEM((NSEQ * D_STATE,), jnp.float32),
            pltpu.VMEM((L,), jnp.int32),
            pltpu.VMEM((L, D_PAIR), jnp.float32),
            pltpu.VMEM((L, D_PAIR), jnp.float32),
            pltpu.VMEM((D_STATE, 128), jnp.float32),
            pltpu.SemaphoreType.DMA,
            pltpu.SemaphoreType.DMA,
        ],
    )
    rawtbl = jnp.concatenate([emb_left, emb_right, pos_emb])
    return kern(seq.reshape(L), idx.reshape(L), rawtbl,
                emb_state.reshape(-1))


_N_BLK = 8


def _tc_body(seq_ref, msa_ref, w_ref, b_ref, q_ref, out_ref, qrow):
    n = pl.program_id(0)

    @pl.when(n == 0)
    def _():
        seq = seq_ref[...]
        onehot_t = (jnp.broadcast_to(seq, (NSEQ, L))
                    == lax.broadcasted_iota(jnp.int32, (NSEQ, L), 0)
                    ).astype(jnp.float32)
        qrow[...] = (lax.dot_general(onehot_t, q_ref[...],
                                     (((0,), (0,)), ((), ())),
                                     preferred_element_type=jnp.float32)
                     + b_ref[...])

    for b in range(_N_BLK):
        x = msa_ref[b]
        y = lax.dot_general(x, w_ref[...], (((0,), (0,)), ((), ())),
                            preferred_element_type=jnp.float32)
        out_ref[b] = y + qrow[...]


def _tc_msa(seq2d, msa3t, emb_Wt, emb_b, emb_q):
    grid = (N // _N_BLK,)
    return pl.pallas_call(
        _tc_body,
        grid=grid,
        in_specs=[
            pl.BlockSpec((1, L), lambda n: (0, 0)),
            pl.BlockSpec((_N_BLK, D_INIT, L), lambda n: (n, 0, 0)),
            pl.BlockSpec((D_INIT, D_MSA), lambda n: (0, 0)),
            pl.BlockSpec((1, D_MSA), lambda n: (0, 0)),
            pl.BlockSpec((NSEQ, D_MSA), lambda n: (0, 0)),
        ],
        out_specs=pl.BlockSpec((_N_BLK, L, D_MSA), lambda n: (n, 0, 0)),
        out_shape=jax.ShapeDtypeStruct((N, L, D_MSA), jnp.float32),
        scratch_shapes=[pltpu.VMEM((L, D_MSA), jnp.float32)],
    )(seq2d, msa3t, emb_Wt, emb_b, emb_q)


@jax.jit
def kernel(msa, seq, idx, emb_W, emb_b, emb_q, emb_left, emb_right,
           emb_state, pos_emb):
    seq2 = seq.reshape(1, L).astype(jnp.int32)
    idx2 = idx.reshape(1, L).astype(jnp.int32)

    msa_e = _tc_msa(seq2,
                    msa.reshape(N, L, D_INIT).transpose(0, 2, 1),
                    emb_W.T, emb_b.reshape(1, D_MSA), emb_q)
    pair, statet = _sc_pair_state(seq2, idx2, emb_left, emb_right, pos_emb,
                                  emb_state)

    return (msa_e.reshape(B, N, L, D_MSA),
            pair.reshape(B, L, L, D_PAIR),
            statet.T.reshape(B, L, D_STATE))
